# baseline (speedup 1.0000x reference)
import functools

import jax
import jax.numpy as jnp
from jax import lax
from jax.experimental import pallas as pl
from jax.experimental.pallas import tpu as pltpu
from jax.experimental.pallas import tpu_sc as plsc

_L = 16
_NC = 2
_NS = 16
_NW = _NC * _NS


def kernel(x, pos_table):
    B, T, E = x.shape
    N = B * T
    rows_per_w = N // _NW
    R = 16
    NBUF = 4
    nsteps = rows_per_w // (R * NBUF)
    x2 = x.reshape(N, E)

    mesh = plsc.VectorSubcoreMesh(core_axis_name="c", subcore_axis_name="s")

    @functools.partial(
        pl.kernel,
        mesh=mesh,
        out_type=jax.ShapeDtypeStruct((N, E), jnp.float32),
        scratch_types=[
            pltpu.VMEM((NBUF, R, E), jnp.float32),
            pltpu.VMEM((E,), jnp.float32),
            pltpu.SemaphoreType.DMA,
            pltpu.SemaphoreType.DMA,
        ],
    )
    def sc_add(x_hbm, pt_hbm, out_hbm, buf, pos_v, in_sem, out_sem):
        wid = lax.axis_index("s") * _NC + lax.axis_index("c")
        base = wid * rows_per_w
        b = base // T
        pltpu.sync_copy(pt_hbm.at[b], pos_v)

        def row_body(sl):
            def body(r, carry):
                for c in range(E // _L):
                    sli = pl.ds(c * _L, _L)
                    buf[sl, r, sli] = buf[sl, r, sli] + pos_v[sli]
                return carry
            return body

        def outer(step, carry):
            row0 = base + step * (NBUF * R)
            cps_in = [
                pltpu.async_copy(
                    x_hbm.at[pl.ds(row0 + sl * R, R)], buf.at[sl], in_sem)
                for sl in range(NBUF)
            ]
            cps_out = []
            for sl in range(NBUF):
                cps_in[sl].wait()
                lax.fori_loop(0, R, row_body(sl), 0)
                cps_out.append(pltpu.async_copy(
                    buf.at[sl], out_hbm.at[pl.ds(row0 + sl * R, R)], out_sem))
            for cp in cps_out:
                cp.wait()
            return carry

        lax.fori_loop(0, nsteps, outer, 0)

    out = sc_add(x2, pos_table)
    return out.reshape(B, T, E)
